# baseline (speedup 1.0000x reference)
import functools

import jax
import jax.numpy as jnp
from jax import lax
from jax.experimental import pallas as pl
from jax.experimental.pallas import tpu as pltpu
from jax.experimental.pallas import tpu_sc as plsc

HIDDEN = 2048
N_EXPERTS = 16
TOP_K = 2
TILE = 1024

_SC_INFO = plsc.get_sparse_core_info()
_NC = _SC_INFO.num_cores
_NS = _SC_INFO.num_subcores
_L = _SC_INFO.num_lanes
_NW = _NC * _NS


def _tc_body(x_ref, w_ref, w_out_ref):
    lt = jax.lax.dot_general(
        w_ref[...], x_ref[...],
        dimension_numbers=(((1,), (1,)), ((), ())),
        preferred_element_type=jnp.float32,
    )
    m = jnp.max(lt, axis=0, keepdims=True)
    e = jnp.exp(lt - m)
    s = jnp.sum(e, axis=0, keepdims=True)
    w_out_ref[...] = e * (1.0 / s)


def _make_sc_route(n):
    chunk = n // _NW
    mesh = plsc.VectorSubcoreMesh(core_axis_name="c", subcore_axis_name="s")

    @functools.partial(
        pl.kernel, mesh=mesh,
        out_type=[
            jax.ShapeDtypeStruct((TOP_K, n), jnp.float32),
            jax.ShapeDtypeStruct((TOP_K, n), jnp.int32),
        ],
        scratch_types=[
            pltpu.VMEM((N_EXPERTS, chunk), jnp.float32),
            pltpu.VMEM((TOP_K, chunk), jnp.float32),
            pltpu.VMEM((TOP_K, chunk), jnp.int32),
            pltpu.SemaphoreType.DMA,
        ],
    )
    def sc_route(wt_hbm, twt_hbm, tet_hbm, wv, twv, tev, sem):
        wid = lax.axis_index("s") * _NC + lax.axis_index("c")
        base = wid * chunk
        pltpu.async_copy(wt_hbm.at[:, pl.ds(base, chunk)], wv, sem).wait()

        def body(g, carry):
            off = g * _L
            rows = [wv[e, pl.ds(off, _L)] for e in range(N_EXPERTS)]
            m1 = rows[0]
            i1 = jnp.zeros((_L,), jnp.int32)
            for e in range(1, N_EXPERTS):
                c = rows[e] > m1
                m1 = jnp.where(c, rows[e], m1)
                i1 = jnp.where(c, jnp.full((_L,), e, jnp.int32), i1)
            m2 = jnp.full((_L,), -jnp.inf, jnp.float32)
            i2 = jnp.zeros((_L,), jnp.int32)
            for e in range(N_EXPERTS):
                c = jnp.logical_and(i1 != e, rows[e] > m2)
                m2 = jnp.where(c, rows[e], m2)
                i2 = jnp.where(c, jnp.full((_L,), e, jnp.int32), i2)
            twv[0, pl.ds(off, _L)] = m1
            twv[1, pl.ds(off, _L)] = m2
            tev[0, pl.ds(off, _L)] = i1
            tev[1, pl.ds(off, _L)] = i2
            return carry

        jax.lax.fori_loop(0, chunk // _L, body, 0)
        pltpu.sync_copy(twv, twt_hbm.at[:, pl.ds(base, chunk)])
        pltpu.sync_copy(tev, tet_hbm.at[:, pl.ds(base, chunk)])

    return sc_route


def kernel(x, W):
    n = x.shape[0]
    grid = (n // TILE,)
    weights_t = pl.pallas_call(
        _tc_body,
        grid=grid,
        in_specs=[
            pl.BlockSpec((TILE, HIDDEN), lambda i: (i, 0)),
            pl.BlockSpec((N_EXPERTS, HIDDEN), lambda i: (0, 0)),
        ],
        out_specs=pl.BlockSpec((N_EXPERTS, TILE), lambda i: (0, i)),
        out_shape=jax.ShapeDtypeStruct((N_EXPERTS, n), jnp.float32),
        compiler_params=pltpu.CompilerParams(
            dimension_semantics=("parallel",),
        ),
    )(x, W)
    top_w_t, top_e_t = _make_sc_route(n)(weights_t)
    return (
        weights_t.T,
        top_w_t.T,
        top_e_t.T.astype(jnp.int64),
    )
